# Initial kernel scaffold; baseline (speedup 1.0000x reference)
import jax
import jax.numpy as jnp
from jax.experimental import pallas as pl


def kernel(x, embedding_table, possitional_emb):
    raise NotImplementedError("write your pallas kernel here")


# speedup vs baseline: 1.4383x; 1.4383x over previous
import functools

import jax
import jax.numpy as jnp
from jax import lax
from jax.experimental import pallas as pl
from jax.experimental.pallas import tpu as pltpu
from jax.experimental.pallas import tpu_sc as plsc

B = 4096
L = 200
D = 32
NW = 32
LG = 4
CB = 256
NCHUNK = B // CB
ITEMS = (L // LG) * NCHUNK
PER_W = ITEMS // NW
IDX_ROWS = LG * CB // 128

_mesh = plsc.VectorSubcoreMesh(core_axis_name="c", subcore_axis_name="s")


@functools.partial(
    pl.kernel,
    out_type=jax.ShapeDtypeStruct((B, L * D), jnp.float32),
    mesh=_mesh,
    scratch_types=[
        pltpu.VMEM((IDX_ROWS, 128), jnp.int32),
        pltpu.VMEM((LG, CB, D), jnp.float32),
        pltpu.VMEM((CB, LG * D), jnp.float32),
        pltpu.VMEM((L, D), jnp.float32),
        pltpu.SemaphoreType.DMA,
    ],
    compiler_params=pltpu.CompilerParams(use_tc_tiling_on_sc=False),
)
def _emb_lookup(x_hbm, table_hbm, pos_hbm, out_hbm,
                idx_v, tmp_v, rows_v, pos_v, gsem):
    wid = lax.axis_index("s") * 2 + lax.axis_index("c")
    pltpu.sync_copy(pos_hbm, pos_v)

    def item_body(j, carry):
        m = wid * PER_W + j
        lg = m // NCHUNK
        c = m % NCHUNK
        pltpu.sync_copy(x_hbm.at[pl.ds(m * IDX_ROWS, IDX_ROWS)], idx_v)
        descs = [
            pltpu.async_copy(table_hbm.at[idx_v.at[k]],
                             tmp_v.at[k // 2, pl.ds((k % 2) * 128, 128)],
                             gsem)
            for k in range(IDX_ROWS)
        ]
        for d in descs:
            d.wait()

        for i in range(LG):
            l = lg * LG + i
            pos_lo = pos_v[l, pl.ds(0, 16)]
            pos_hi = pos_v[l, pl.ds(16, 16)]

            def row_body(r, _, i=i, pos_lo=pos_lo, pos_hi=pos_hi):
                rows_v[r, pl.ds(i * D, 16)] = (
                    tmp_v[i, r, pl.ds(0, 16)] + pos_lo)
                rows_v[r, pl.ds(i * D + 16, 16)] = (
                    tmp_v[i, r, pl.ds(16, 16)] + pos_hi)
                return 0

            lax.fori_loop(0, CB, row_body, 0)

        pltpu.sync_copy(rows_v,
                        out_hbm.at[pl.ds(c * CB, CB), pl.ds(lg * 128, 128)])
        return carry

    lax.fori_loop(0, PER_W, item_body, 0)


def kernel(x, embedding_table, possitional_emb):
    xi = (x.T.astype(jnp.int32)
          .reshape(L // LG, LG, NCHUNK, CB)
          .transpose(0, 2, 1, 3)
          .reshape(ITEMS * IDX_ROWS, 128))
    out = _emb_lookup(xi, embedding_table, possitional_emb)
    return out.reshape(B, L, D)
